# Initial kernel scaffold; baseline (speedup 1.0000x reference)
import jax
import jax.numpy as jnp
from jax.experimental import pallas as pl


def kernel(position, velocity, force, edge_index, W0, b0, W1, b1, Wl, bl):
    raise NotImplementedError("write your pallas kernel here")


# speedup vs baseline: 5.8996x; 5.8996x over previous
import functools

import jax
import jax.numpy as jnp
from jax import lax
from jax.experimental import pallas as pl
from jax.experimental.pallas import tpu as pltpu
from jax.experimental.pallas import tpu_sc as plsc

N = 100000
E = 1600000
HID = 32
NC = 2
NS = 16
LANES = 16
HALF = N // NC
ZPT = 3128
HALF_P = ZPT * NS
EPT = E // NS
SUP = 4000
CH = 80
N_SUP = EPT // SUP
N_CH = SUP // CH
CPO = 400
NCPO = HALF // CPO
ZTAIL = ZPT - 7 * CPO


def _seg_sum_body(h_hbm, src_hbm, dst_hbm, agg_hbm,
                  src_sb, dst_sb, rowbuf, didx, obuf, agg_sh, sem):
    c = lax.axis_index("c")
    s = lax.axis_index("s")
    base_node = c * HALF
    ebase = s * EPT
    dummy = HALF + s

    z16 = jnp.zeros((LANES,), jnp.float32)

    def _zero_row(i, _):
        obuf[i, pl.ds(0, LANES)] = z16
        obuf[i, pl.ds(LANES, LANES)] = z16
        return 0

    lax.fori_loop(0, CPO, _zero_row, 0)
    zbase = pl.multiple_of(s * ZPT, 8)
    for k in range(7):
        pltpu.sync_copy(obuf, agg_sh.at[pl.ds(zbase + k * CPO, CPO)])
    pltpu.sync_copy(obuf.at[pl.ds(0, ZTAIL)],
                    agg_sh.at[pl.ds(zbase + 7 * CPO, ZTAIL)])
    plsc.subcore_barrier()

    def _sup_body(u, _):
        eoff = pl.multiple_of(ebase + u * SUP, 8)
        pltpu.sync_copy(src_hbm.at[pl.ds(eoff, SUP)], src_sb)
        pltpu.sync_copy(dst_hbm.at[pl.ds(eoff, SUP)], dst_sb)

        def _ch_body(j, _):
            off = pl.multiple_of(j * CH, 8)
            for q in range(CH // LANES):
                d = dst_sb[pl.ds(off + q * LANES, LANES)]
                inr = (d >= base_node) & (d < base_node + HALF)
                didx[pl.ds(q * LANES, LANES)] = jnp.where(
                    inr, d - base_node, dummy)
            pltpu.async_copy(
                h_hbm.at[src_sb.at[pl.ds(off, CH)]], rowbuf, sem).wait()
            pltpu.sync_copy(rowbuf, agg_sh.at[didx], add=True)
            return 0

        lax.fori_loop(0, N_CH, _ch_body, 0)
        return 0

    lax.fori_loop(0, N_SUP, _sup_body, 0)
    plsc.subcore_barrier()

    def _copy_chunk(chunk):
        r = pl.multiple_of(chunk * CPO, 8)
        g = pl.multiple_of(base_node + chunk * CPO, 8)
        pltpu.sync_copy(agg_sh.at[pl.ds(r, CPO)], obuf)
        pltpu.sync_copy(obuf, agg_hbm.at[pl.ds(g, CPO)])

    for k in range(7):
        _copy_chunk(s + k * NS)

    @pl.when(s < NCPO - 7 * NS)
    def _():
        _copy_chunk(s + 7 * NS)


def _segment_sum_sc(h, src, dst):
    mesh = plsc.VectorSubcoreMesh(core_axis_name="c", subcore_axis_name="s")
    f = pl.kernel(
        _seg_sum_body,
        out_type=jax.ShapeDtypeStruct((N, HID), jnp.float32),
        mesh=mesh,
        scratch_types=[
            pltpu.VMEM((SUP,), jnp.int32),
            pltpu.VMEM((SUP,), jnp.int32),
            pltpu.VMEM((CH, HID), jnp.float32),
            pltpu.VMEM((CH,), jnp.int32),
            pltpu.VMEM((CPO, HID), jnp.float32),
            pltpu.VMEM_SHARED((HALF_P, HID), jnp.float32),
            pltpu.SemaphoreType.DMA,
        ],
        compiler_params=pltpu.CompilerParams(use_tc_tiling_on_sc=False),
    )
    return f(h, src, dst)


BN = 2000


def _mm_body(x_ref, w_ref, b_ref, o_ref):
    o_ref[...] = jnp.dot(x_ref[...], w_ref[...],
                         preferred_element_type=jnp.float32) + b_ref[...]


def _matmul_bias(x, w, b):
    k = x.shape[1]
    m = w.shape[1]
    return pl.pallas_call(
        _mm_body,
        grid=(N // BN,),
        in_specs=[
            pl.BlockSpec((BN, k), lambda i: (i, 0)),
            pl.BlockSpec((k, m), lambda i: (0, 0)),
            pl.BlockSpec((1, m), lambda i: (0, 0)),
        ],
        out_specs=pl.BlockSpec((BN, m), lambda i: (i, 0)),
        out_shape=jax.ShapeDtypeStruct((N, m), jnp.float32),
    )(x, w, b)


def _gconv_out_body(h_ref, a_ref, w_ref, b_ref, o_ref):
    x = jnp.maximum(h_ref[...] + a_ref[...], 0.0)
    o_ref[...] = jnp.dot(x, w_ref[...],
                         preferred_element_type=jnp.float32) + b_ref[...]


def _relu_add_matmul(h, a, w, b):
    k = w.shape[0]
    m = w.shape[1]
    return pl.pallas_call(
        _gconv_out_body,
        grid=(N // BN,),
        in_specs=[
            pl.BlockSpec((BN, k), lambda i: (i, 0)),
            pl.BlockSpec((BN, k), lambda i: (i, 0)),
            pl.BlockSpec((k, m), lambda i: (0, 0)),
            pl.BlockSpec((1, m), lambda i: (0, 0)),
        ],
        out_specs=pl.BlockSpec((BN, m), lambda i: (i, 0)),
        out_shape=jax.ShapeDtypeStruct((N, m), jnp.float32),
    )(h, a, w, b)


def kernel(position, velocity, force, edge_index, W0, b0, W1, b1, Wl, bl):
    x = jnp.concatenate([position, velocity, force], axis=1)
    src = edge_index[0]
    dst = edge_index[1]
    h0 = _matmul_bias(x, W0, b0.reshape(1, HID))
    agg0 = _segment_sum_sc(h0, src, dst)
    h1 = _relu_add_matmul(h0, agg0, W1, b1.reshape(1, HID))
    agg1 = _segment_sum_sc(h1, src, dst)
    return _relu_add_matmul(h1, agg1, Wl, bl.reshape(1, bl.shape[0]))


# speedup vs baseline: 9.5844x; 1.6246x over previous
import functools

import jax
import jax.numpy as jnp
from jax import lax
from jax.experimental import pallas as pl
from jax.experimental.pallas import tpu as pltpu
from jax.experimental.pallas import tpu_sc as plsc

N = 100000
E = 1600000
HID = 32
NC = 2
NS = 16
LANES = 16
HALF = N // NC
ZPT = 3128
HALF_P = ZPT * NS
EPT = E // NS
SUP = 4000
CH = 80
N_SUP = EPT // SUP
N_CH = SUP // CH
CPO = 400
NCPO = HALF // CPO
ZTAIL = ZPT - 7 * CPO


def _seg_sum_body(h_hbm, src_hbm, dst_hbm, agg_hbm,
                  src_sb, dst_sb, rowbuf0, rowbuf1, didx0, didx1, obuf,
                  agg_sh, sem0, sem1):
    c = lax.axis_index("c")
    s = lax.axis_index("s")
    base_node = c * HALF
    ebase = s * EPT
    dummy = HALF + s
    rowbufs = (rowbuf0, rowbuf1)
    didxs = (didx0, didx1)
    sems = (sem0, sem1)

    z16 = jnp.zeros((LANES,), jnp.float32)

    def _zero_row(i, _):
        obuf[i, pl.ds(0, LANES)] = z16
        obuf[i, pl.ds(LANES, LANES)] = z16
        return 0

    lax.fori_loop(0, CPO, _zero_row, 0)
    zbase = pl.multiple_of(s * ZPT, 8)
    for k in range(7):
        pltpu.sync_copy(obuf, agg_sh.at[pl.ds(zbase + k * CPO, CPO)])
    pltpu.sync_copy(obuf.at[pl.ds(0, ZTAIL)],
                    agg_sh.at[pl.ds(zbase + 7 * CPO, ZTAIL)])
    plsc.subcore_barrier()

    def _prep_issue(j, b):
        off = pl.multiple_of(j * CH, 8)
        for q in range(CH // LANES):
            d = dst_sb[pl.ds(off + q * LANES, LANES)]
            inr = (d >= base_node) & (d < base_node + HALF)
            didxs[b][pl.ds(q * LANES, LANES)] = jnp.where(
                inr, d - base_node, dummy)
        pltpu.async_copy(
            h_hbm.at[src_sb.at[pl.ds(off, CH)]], rowbufs[b], sems[b])

    def _wait_scatter(b):
        pltpu.make_async_copy(
            h_hbm.at[src_sb.at[pl.ds(0, CH)]], rowbufs[b], sems[b]).wait()
        pltpu.sync_copy(rowbufs[b], agg_sh.at[didxs[b]], add=True)

    def _sup_body(u, _):
        eoff = pl.multiple_of(ebase + u * SUP, 8)
        pltpu.sync_copy(src_hbm.at[pl.ds(eoff, SUP)], src_sb)
        pltpu.sync_copy(dst_hbm.at[pl.ds(eoff, SUP)], dst_sb)

        _prep_issue(0, 0)
        _prep_issue(1, 1)

        def _pair_body(p, _):
            for b in range(2):
                j = 2 * p + b
                _wait_scatter(b)

                @pl.when(j + 2 < N_CH)
                def _():
                    _prep_issue(j + 2, b)
            return 0

        lax.fori_loop(0, N_CH // 2, _pair_body, 0)
        return 0

    lax.fori_loop(0, N_SUP, _sup_body, 0)
    plsc.subcore_barrier()

    def _copy_chunk(chunk):
        r = pl.multiple_of(chunk * CPO, 8)
        g = pl.multiple_of(base_node + chunk * CPO, 8)
        pltpu.sync_copy(agg_sh.at[pl.ds(r, CPO)], obuf)
        pltpu.sync_copy(obuf, agg_hbm.at[pl.ds(g, CPO)])

    for k in range(7):
        _copy_chunk(s + k * NS)

    @pl.when(s < NCPO - 7 * NS)
    def _():
        _copy_chunk(s + 7 * NS)


def _segment_sum_sc(h, src, dst):
    mesh = plsc.VectorSubcoreMesh(core_axis_name="c", subcore_axis_name="s")
    f = pl.kernel(
        _seg_sum_body,
        out_type=jax.ShapeDtypeStruct((N, HID), jnp.float32),
        mesh=mesh,
        scratch_types=[
            pltpu.VMEM((SUP,), jnp.int32),
            pltpu.VMEM((SUP,), jnp.int32),
            pltpu.VMEM((CH, HID), jnp.float32),
            pltpu.VMEM((CH, HID), jnp.float32),
            pltpu.VMEM((CH,), jnp.int32),
            pltpu.VMEM((CH,), jnp.int32),
            pltpu.VMEM((CPO, HID), jnp.float32),
            pltpu.VMEM_SHARED((HALF_P, HID), jnp.float32),
            pltpu.SemaphoreType.DMA,
            pltpu.SemaphoreType.DMA,
        ],
        compiler_params=pltpu.CompilerParams(use_tc_tiling_on_sc=False),
    )
    return f(h, src, dst)


BN = 2000


def _mm_body(x_ref, w_ref, b_ref, o_ref):
    o_ref[...] = jnp.dot(x_ref[...], w_ref[...],
                         preferred_element_type=jnp.float32) + b_ref[...]


def _matmul_bias(x, w, b):
    k = x.shape[1]
    m = w.shape[1]
    return pl.pallas_call(
        _mm_body,
        grid=(N // BN,),
        in_specs=[
            pl.BlockSpec((BN, k), lambda i: (i, 0)),
            pl.BlockSpec((k, m), lambda i: (0, 0)),
            pl.BlockSpec((1, m), lambda i: (0, 0)),
        ],
        out_specs=pl.BlockSpec((BN, m), lambda i: (i, 0)),
        out_shape=jax.ShapeDtypeStruct((N, m), jnp.float32),
    )(x, w, b)


def _gconv_out_body(h_ref, a_ref, w_ref, b_ref, o_ref):
    x = jnp.maximum(h_ref[...] + a_ref[...], 0.0)
    o_ref[...] = jnp.dot(x, w_ref[...],
                         preferred_element_type=jnp.float32) + b_ref[...]


def _relu_add_matmul(h, a, w, b):
    k = w.shape[0]
    m = w.shape[1]
    return pl.pallas_call(
        _gconv_out_body,
        grid=(N // BN,),
        in_specs=[
            pl.BlockSpec((BN, k), lambda i: (i, 0)),
            pl.BlockSpec((BN, k), lambda i: (i, 0)),
            pl.BlockSpec((k, m), lambda i: (0, 0)),
            pl.BlockSpec((1, m), lambda i: (0, 0)),
        ],
        out_specs=pl.BlockSpec((BN, m), lambda i: (i, 0)),
        out_shape=jax.ShapeDtypeStruct((N, m), jnp.float32),
    )(h, a, w, b)


def kernel(position, velocity, force, edge_index, W0, b0, W1, b1, Wl, bl):
    x = jnp.concatenate([position, velocity, force], axis=1)
    src = edge_index[0]
    dst = edge_index[1]
    h0 = _matmul_bias(x, W0, b0.reshape(1, HID))
    agg0 = _segment_sum_sc(h0, src, dst)
    h1 = _relu_add_matmul(h0, agg0, W1, b1.reshape(1, HID))
    agg1 = _segment_sum_sc(h1, src, dst)
    return _relu_add_matmul(h1, agg1, Wl, bl.reshape(1, bl.shape[0]))


# speedup vs baseline: 9.9284x; 1.0359x over previous
import functools

import jax
import jax.numpy as jnp
from jax import lax
from jax.experimental import pallas as pl
from jax.experimental.pallas import tpu as pltpu
from jax.experimental.pallas import tpu_sc as plsc

N = 100000
E = 1600000
FIN = 16
HID = 32
NC = 2
NS = 16
LANES = 16
HALF = N // NC
ZPT = 3128
HALF_P = ZPT * NS
EPT = E // NS
SUP = 4000
CH = 80
N_SUP = EPT // SUP
N_CH = SUP // CH
CPO = 400
NCPO = HALF // CPO
ZTAIL = ZPT - 7 * CPO


def _make_seg_body(hid):
    def body(h_hbm, src_hbm, dst_hbm, agg_hbm,
             src_sb, dst_sb, rowbuf0, rowbuf1, didx0, didx1, obuf,
             agg_sh, sem0, sem1):
        c = lax.axis_index("c")
        s = lax.axis_index("s")
        base_node = c * HALF
        ebase = s * EPT
        dummy = HALF + s
        rowbufs = (rowbuf0, rowbuf1)
        didxs = (didx0, didx1)
        sems = (sem0, sem1)

        z16 = jnp.zeros((LANES,), jnp.float32)

        def _zero_row(i, _):
            for q in range(hid // LANES):
                obuf[i, pl.ds(q * LANES, LANES)] = z16
            return 0

        lax.fori_loop(0, CPO, _zero_row, 0)
        zbase = pl.multiple_of(s * ZPT, 8)
        for k in range(7):
            pltpu.sync_copy(obuf, agg_sh.at[pl.ds(zbase + k * CPO, CPO)])
        pltpu.sync_copy(obuf.at[pl.ds(0, ZTAIL)],
                        agg_sh.at[pl.ds(zbase + 7 * CPO, ZTAIL)])
        plsc.subcore_barrier()

        def _prep_issue(j, b):
            off = pl.multiple_of(j * CH, 8)
            for q in range(CH // LANES):
                d = dst_sb[pl.ds(off + q * LANES, LANES)]
                inr = (d >= base_node) & (d < base_node + HALF)
                didxs[b][pl.ds(q * LANES, LANES)] = jnp.where(
                    inr, d - base_node, dummy)
            pltpu.async_copy(
                h_hbm.at[src_sb.at[pl.ds(off, CH)]], rowbufs[b], sems[b])

        def _wait_scatter(b):
            pltpu.make_async_copy(
                h_hbm.at[src_sb.at[pl.ds(0, CH)]], rowbufs[b],
                sems[b]).wait()
            pltpu.sync_copy(rowbufs[b], agg_sh.at[didxs[b]], add=True)

        def _sup_body(u, _):
            eoff = pl.multiple_of(ebase + u * SUP, 8)
            pltpu.sync_copy(src_hbm.at[pl.ds(eoff, SUP)], src_sb)
            pltpu.sync_copy(dst_hbm.at[pl.ds(eoff, SUP)], dst_sb)

            _prep_issue(0, 0)
            _prep_issue(1, 1)

            def _pair_body(p, _):
                for b in range(2):
                    j = 2 * p + b
                    _wait_scatter(b)

                    @pl.when(j + 2 < N_CH)
                    def _():
                        _prep_issue(j + 2, b)
                return 0

            lax.fori_loop(0, N_CH // 2, _pair_body, 0)
            return 0

        lax.fori_loop(0, N_SUP, _sup_body, 0)
        plsc.subcore_barrier()

        def _copy_chunk(chunk):
            r = pl.multiple_of(chunk * CPO, 8)
            g = pl.multiple_of(base_node + chunk * CPO, 8)
            pltpu.sync_copy(agg_sh.at[pl.ds(r, CPO)], obuf)
            pltpu.sync_copy(obuf, agg_hbm.at[pl.ds(g, CPO)])

        for k in range(7):
            _copy_chunk(s + k * NS)

        @pl.when(s < NCPO - 7 * NS)
        def _():
            _copy_chunk(s + 7 * NS)

    return body


def _segment_sum_sc(h, src, dst, hid):
    mesh = plsc.VectorSubcoreMesh(core_axis_name="c", subcore_axis_name="s")
    f = pl.kernel(
        _make_seg_body(hid),
        out_type=jax.ShapeDtypeStruct((N, hid), jnp.float32),
        mesh=mesh,
        scratch_types=[
            pltpu.VMEM((SUP,), jnp.int32),
            pltpu.VMEM((SUP,), jnp.int32),
            pltpu.VMEM((CH, hid), jnp.float32),
            pltpu.VMEM((CH, hid), jnp.float32),
            pltpu.VMEM((CH,), jnp.int32),
            pltpu.VMEM((CH,), jnp.int32),
            pltpu.VMEM((CPO, hid), jnp.float32),
            pltpu.VMEM_SHARED((HALF_P, hid), jnp.float32),
            pltpu.SemaphoreType.DMA,
            pltpu.SemaphoreType.DMA,
        ],
        compiler_params=pltpu.CompilerParams(use_tc_tiling_on_sc=False),
    )
    return f(h, src, dst)


BN = 2000


def _layer1_body(x_ref, s_ref, w0_ref, w1_ref, b1_ref, o_ref):
    x1 = jnp.maximum(
        jnp.dot(x_ref[...] + s_ref[...], w0_ref[...],
                preferred_element_type=jnp.float32), 0.0)
    o_ref[...] = jnp.dot(x1, w1_ref[...],
                         preferred_element_type=jnp.float32) + b1_ref[...]


def _layer1_tc(x_ext, s0, w0ext, w1, b1):
    return pl.pallas_call(
        _layer1_body,
        grid=(N // BN,),
        in_specs=[
            pl.BlockSpec((BN, FIN), lambda i: (i, 0)),
            pl.BlockSpec((BN, FIN), lambda i: (i, 0)),
            pl.BlockSpec((FIN, HID), lambda i: (0, 0)),
            pl.BlockSpec((HID, HID), lambda i: (0, 0)),
            pl.BlockSpec((1, HID), lambda i: (0, 0)),
        ],
        out_specs=pl.BlockSpec((BN, HID), lambda i: (i, 0)),
        out_shape=jax.ShapeDtypeStruct((N, HID), jnp.float32),
    )(x_ext, s0, w0ext, w1, b1)


def _gconv_out_body(h_ref, a_ref, w_ref, b_ref, o_ref):
    x = jnp.maximum(h_ref[...] + a_ref[...], 0.0)
    o_ref[...] = jnp.dot(x, w_ref[...],
                         preferred_element_type=jnp.float32) + b_ref[...]


def _relu_add_matmul(h, a, w, b):
    k = w.shape[0]
    m = w.shape[1]
    return pl.pallas_call(
        _gconv_out_body,
        grid=(N // BN,),
        in_specs=[
            pl.BlockSpec((BN, k), lambda i: (i, 0)),
            pl.BlockSpec((BN, k), lambda i: (i, 0)),
            pl.BlockSpec((k, m), lambda i: (0, 0)),
            pl.BlockSpec((1, m), lambda i: (0, 0)),
        ],
        out_specs=pl.BlockSpec((BN, m), lambda i: (i, 0)),
        out_shape=jax.ShapeDtypeStruct((N, m), jnp.float32),
    )(h, a, w, b)


def kernel(position, velocity, force, edge_index, W0, b0, W1, b1, Wl, bl):
    ones = jnp.ones((N, 1), jnp.float32)
    zpad = jnp.zeros((N, FIN - 10), jnp.float32)
    x_ext = jnp.concatenate([position, velocity, force, ones, zpad], axis=1)
    w0ext = jnp.concatenate(
        [W0, b0[None, :], jnp.zeros((FIN - 10, HID), jnp.float32)], axis=0)
    src = edge_index[0]
    dst = edge_index[1]
    s0 = _segment_sum_sc(x_ext, src, dst, FIN)
    h1 = _layer1_tc(x_ext, s0, w0ext, W1, b1.reshape(1, HID))
    a1 = _segment_sum_sc(h1, src, dst, HID)
    return _relu_add_matmul(h1, a1, Wl, bl.reshape(1, bl.shape[0]))


# speedup vs baseline: 10.9617x; 1.1041x over previous
import functools

import jax
import jax.numpy as jnp
from jax import lax
from jax.experimental import pallas as pl
from jax.experimental.pallas import tpu as pltpu
from jax.experimental.pallas import tpu_sc as plsc

N = 100000
E = 1600000
FIN = 16
HID = 32
NC = 2
NS = 16
LANES = 16
HALF = N // NC
ZPT = 3128
HALF_P = ZPT * NS
EPT = E // NS
SUP = 4000
CH = 80
N_SUP = EPT // SUP
N_CH = SUP // CH
CPO = 200
NCPO = HALF // CPO
NZC = ZPT // CPO
ZTAIL = ZPT - NZC * CPO
NBUF = 4


def _make_seg_body(hid):
    def body(h_hbm, src_hbm, dst_hbm, agg_hbm,
             src_sb, dst_sb,
             rowbuf0, rowbuf1, rowbuf2, rowbuf3,
             didx0, didx1, didx2, didx3, obuf, agg_sh,
             g0, g1, g2, g3, s0, s1, s2, s3):
        c = lax.axis_index("c")
        s = lax.axis_index("s")
        base_node = c * HALF
        ebase = s * EPT
        dummy = HALF + s
        rowbufs = (rowbuf0, rowbuf1, rowbuf2, rowbuf3)
        didxs = (didx0, didx1, didx2, didx3)
        gsems = (g0, g1, g2, g3)
        ssems = (s0, s1, s2, s3)

        z16 = jnp.zeros((LANES,), jnp.float32)

        def _zero_row(i, _):
            for q in range(hid // LANES):
                obuf[i, pl.ds(q * LANES, LANES)] = z16
            return 0

        lax.fori_loop(0, CPO, _zero_row, 0)
        zbase = pl.multiple_of(s * ZPT, 8)
        for k in range(NZC):
            pltpu.sync_copy(obuf, agg_sh.at[pl.ds(zbase + k * CPO, CPO)])
        pltpu.sync_copy(obuf.at[pl.ds(0, ZTAIL)],
                        agg_sh.at[pl.ds(zbase + NZC * CPO, ZTAIL)])
        plsc.subcore_barrier()

        def _prep_issue(j, b):
            off = pl.multiple_of(j * CH, 8)
            for q in range(CH // LANES):
                d = dst_sb[pl.ds(off + q * LANES, LANES)]
                inr = (d >= base_node) & (d < base_node + HALF)
                didxs[b][pl.ds(q * LANES, LANES)] = jnp.where(
                    inr, d - base_node, dummy)
            pltpu.async_copy(
                h_hbm.at[src_sb.at[pl.ds(off, CH)]], rowbufs[b], gsems[b])

        def _wait_gather(b):
            pltpu.make_async_copy(
                h_hbm.at[src_sb.at[pl.ds(0, CH)]], rowbufs[b],
                gsems[b]).wait()

        def _wait_scatter(b):
            pltpu.make_async_copy(
                rowbufs[b], agg_sh.at[didxs[b]], ssems[b]).wait()

        def _sup_body(u, _):
            eoff = pl.multiple_of(ebase + u * SUP, 8)
            pltpu.sync_copy(src_hbm.at[pl.ds(eoff, SUP)], src_sb)
            pltpu.sync_copy(dst_hbm.at[pl.ds(eoff, SUP)], dst_sb)

            _prep_issue(0, 0)
            _prep_issue(1, 1)

            def _quad_body(p, _):
                for b in range(NBUF):
                    j = 4 * p + b

                    @pl.when(j < N_CH)
                    def _():
                        _wait_gather(b)
                        pltpu.async_copy(
                            rowbufs[b], agg_sh.at[didxs[b]], ssems[b],
                            add=True)
                        jn = j + 2

                        @pl.when(jn < N_CH)
                        def _():
                            bn = (b + 2) % NBUF

                            @pl.when(j >= 2)
                            def _():
                                _wait_scatter(bn)

                            _prep_issue(jn, bn)
                return 0

            lax.fori_loop(0, (N_CH + NBUF - 1) // NBUF, _quad_body, 0)
            for jj in range(N_CH - NBUF, N_CH):
                _wait_scatter(jj % NBUF)
            return 0

        lax.fori_loop(0, N_SUP, _sup_body, 0)
        plsc.subcore_barrier()

        def _copy_chunk(chunk):
            r = pl.multiple_of(chunk * CPO, 8)
            g = pl.multiple_of(base_node + chunk * CPO, 8)
            pltpu.sync_copy(agg_sh.at[pl.ds(r, CPO)], obuf)
            pltpu.sync_copy(obuf, agg_hbm.at[pl.ds(g, CPO)])

        for k in range(NCPO // NS):
            _copy_chunk(s + k * NS)

        @pl.when(s < NCPO - (NCPO // NS) * NS)
        def _():
            _copy_chunk(s + (NCPO // NS) * NS)

    return body


def _segment_sum_sc(h, src, dst, hid):
    mesh = plsc.VectorSubcoreMesh(core_axis_name="c", subcore_axis_name="s")
    f = pl.kernel(
        _make_seg_body(hid),
        out_type=jax.ShapeDtypeStruct((N, hid), jnp.float32),
        mesh=mesh,
        scratch_types=(
            [pltpu.VMEM((SUP,), jnp.int32)] * 2
            + [pltpu.VMEM((CH, hid), jnp.float32)] * NBUF
            + [pltpu.VMEM((CH,), jnp.int32)] * NBUF
            + [pltpu.VMEM((CPO, hid), jnp.float32)]
            + [pltpu.VMEM_SHARED((HALF_P, hid), jnp.float32)]
            + [pltpu.SemaphoreType.DMA] * (2 * NBUF)
        ),
        compiler_params=pltpu.CompilerParams(use_tc_tiling_on_sc=False),
    )
    return f(h, src, dst)


BN = 2000


def _layer1_body(x_ref, s_ref, w0_ref, w1_ref, b1_ref, o_ref):
    x1 = jnp.maximum(
        jnp.dot(x_ref[...] + s_ref[...], w0_ref[...],
                preferred_element_type=jnp.float32), 0.0)
    o_ref[...] = jnp.dot(x1, w1_ref[...],
                         preferred_element_type=jnp.float32) + b1_ref[...]


def _layer1_tc(x_ext, s0, w0ext, w1, b1):
    return pl.pallas_call(
        _layer1_body,
        grid=(N // BN,),
        in_specs=[
            pl.BlockSpec((BN, FIN), lambda i: (i, 0)),
            pl.BlockSpec((BN, FIN), lambda i: (i, 0)),
            pl.BlockSpec((FIN, HID), lambda i: (0, 0)),
            pl.BlockSpec((HID, HID), lambda i: (0, 0)),
            pl.BlockSpec((1, HID), lambda i: (0, 0)),
        ],
        out_specs=pl.BlockSpec((BN, HID), lambda i: (i, 0)),
        out_shape=jax.ShapeDtypeStruct((N, HID), jnp.float32),
    )(x_ext, s0, w0ext, w1, b1)


def _gconv_out_body(h_ref, a_ref, w_ref, b_ref, o_ref):
    x = jnp.maximum(h_ref[...] + a_ref[...], 0.0)
    o_ref[...] = jnp.dot(x, w_ref[...],
                         preferred_element_type=jnp.float32) + b_ref[...]


def _relu_add_matmul(h, a, w, b):
    k = w.shape[0]
    m = w.shape[1]
    return pl.pallas_call(
        _gconv_out_body,
        grid=(N // BN,),
        in_specs=[
            pl.BlockSpec((BN, k), lambda i: (i, 0)),
            pl.BlockSpec((BN, k), lambda i: (i, 0)),
            pl.BlockSpec((k, m), lambda i: (0, 0)),
            pl.BlockSpec((1, m), lambda i: (0, 0)),
        ],
        out_specs=pl.BlockSpec((BN, m), lambda i: (i, 0)),
        out_shape=jax.ShapeDtypeStruct((N, m), jnp.float32),
    )(h, a, w, b)


def kernel(position, velocity, force, edge_index, W0, b0, W1, b1, Wl, bl):
    ones = jnp.ones((N, 1), jnp.float32)
    zpad = jnp.zeros((N, FIN - 10), jnp.float32)
    x_ext = jnp.concatenate([position, velocity, force, ones, zpad], axis=1)
    w0ext = jnp.concatenate(
        [W0, b0[None, :], jnp.zeros((FIN - 10, HID), jnp.float32)], axis=0)
    src = edge_index[0]
    dst = edge_index[1]
    s0 = _segment_sum_sc(x_ext, src, dst, FIN)
    h1 = _layer1_tc(x_ext, s0, w0ext, W1, b1.reshape(1, HID))
    a1 = _segment_sum_sc(h1, src, dst, HID)
    return _relu_add_matmul(h1, a1, Wl, bl.reshape(1, bl.shape[0]))


# speedup vs baseline: 11.9573x; 1.0908x over previous
import functools

import jax
import jax.numpy as jnp
from jax import lax
from jax.experimental import pallas as pl
from jax.experimental.pallas import tpu as pltpu
from jax.experimental.pallas import tpu_sc as plsc

N = 100000
E = 1600000
FIN = 16
HID = 32
NC = 2
NS = 16
LANES = 16
HALF = N // NC
ZPT = 3128
HALF_P = ZPT * NS
EPT = E // NS
SUP = 4000
CH = 80
N_SUP = EPT // SUP
N_CH = SUP // CH
CPO = 200
NCPO = HALF // CPO
NZC = ZPT // CPO
ZTAIL = ZPT - NZC * CPO
NBUF = 4


def _make_seg_body(hid, full_n):
    own = N if full_n else HALF
    alloc = 100096 if full_n else HALF_P
    zpt = alloc // NS
    ept = E // (NC * NS) if full_n else EPT
    sup = 2000 if full_n else SUP
    n_sup = ept // sup
    n_ch = sup // CH
    cpo = 400 if full_n else CPO
    nzc = zpt // cpo
    ztail = zpt - nzc * cpo
    ncpo = own // cpo
    rounds = ncpo // NS

    def body(h_hbm, src_hbm, dst_hbm, agg_hbm,
             src_sb, dst_sb,
             rowbuf0, rowbuf1, rowbuf2, rowbuf3,
             didx0, didx1, didx2, didx3, obuf, agg_sh,
             g0, g1, g2, g3, s0, s1, s2, s3):
        c = lax.axis_index("c")
        s = lax.axis_index("s")
        base_node = c * HALF
        if full_n:
            ebase = c * (E // NC) + s * ept
        else:
            ebase = s * ept
        dummy = HALF + s
        rowbufs = (rowbuf0, rowbuf1, rowbuf2, rowbuf3)
        didxs = (didx0, didx1, didx2, didx3)
        gsems = (g0, g1, g2, g3)
        ssems = (s0, s1, s2, s3)

        z16 = jnp.zeros((LANES,), jnp.float32)

        def _zero_row(i, _):
            for q in range(hid // LANES):
                obuf[i, pl.ds(q * LANES, LANES)] = z16
            return 0

        lax.fori_loop(0, cpo, _zero_row, 0)
        zbase = pl.multiple_of(s * zpt, 8)
        for k in range(nzc):
            pltpu.sync_copy(obuf, agg_sh.at[pl.ds(zbase + k * cpo, cpo)])
        if ztail:
            pltpu.sync_copy(obuf.at[pl.ds(0, ztail)],
                            agg_sh.at[pl.ds(zbase + nzc * cpo, ztail)])
        plsc.subcore_barrier()

        def _prep_issue(j, b):
            off = pl.multiple_of(j * CH, 8)
            for q in range(CH // LANES):
                d = dst_sb[pl.ds(off + q * LANES, LANES)]
                if full_n:
                    didxs[b][pl.ds(q * LANES, LANES)] = d
                else:
                    inr = (d >= base_node) & (d < base_node + HALF)
                    didxs[b][pl.ds(q * LANES, LANES)] = jnp.where(
                        inr, d - base_node, dummy)
            pltpu.async_copy(
                h_hbm.at[src_sb.at[pl.ds(off, CH)]], rowbufs[b], gsems[b])

        def _wait_gather(b):
            pltpu.make_async_copy(
                h_hbm.at[src_sb.at[pl.ds(0, CH)]], rowbufs[b],
                gsems[b]).wait()

        def _wait_scatter(b):
            pltpu.make_async_copy(
                rowbufs[b], agg_sh.at[didxs[b]], ssems[b]).wait()

        def _sup_body(u, _):
            eoff = pl.multiple_of(ebase + u * sup, 8)
            pltpu.sync_copy(src_hbm.at[pl.ds(eoff, sup)], src_sb)
            pltpu.sync_copy(dst_hbm.at[pl.ds(eoff, sup)], dst_sb)

            _prep_issue(0, 0)
            _prep_issue(1, 1)

            def _quad_body(p, _):
                for b in range(NBUF):
                    j = 4 * p + b

                    @pl.when(j < n_ch)
                    def _():
                        _wait_gather(b)
                        pltpu.async_copy(
                            rowbufs[b], agg_sh.at[didxs[b]], ssems[b],
                            add=True)
                        jn = j + 2

                        @pl.when(jn < n_ch)
                        def _():
                            bn = (b + 2) % NBUF

                            @pl.when(j >= 2)
                            def _():
                                _wait_scatter(bn)

                            _prep_issue(jn, bn)
                return 0

            lax.fori_loop(0, (n_ch + NBUF - 1) // NBUF, _quad_body, 0)
            for jj in range(n_ch - NBUF, n_ch):
                _wait_scatter(jj % NBUF)
            return 0

        lax.fori_loop(0, n_sup, _sup_body, 0)
        plsc.subcore_barrier()

        def _copy_chunk(chunk):
            r = pl.multiple_of(chunk * cpo, 8)
            pltpu.sync_copy(agg_sh.at[pl.ds(r, cpo)], obuf)
            if full_n:
                pltpu.sync_copy(obuf, agg_hbm.at[c, pl.ds(r, cpo)])
            else:
                g = pl.multiple_of(base_node + chunk * cpo, 8)
                pltpu.sync_copy(obuf, agg_hbm.at[pl.ds(g, cpo)])

        for k in range(rounds):
            _copy_chunk(s + k * NS)

        @pl.when(s < ncpo - rounds * NS)
        def _():
            _copy_chunk(s + rounds * NS)

    return body


def _segment_sum_sc(h, src, dst, hid, full_n):
    mesh = plsc.VectorSubcoreMesh(core_axis_name="c", subcore_axis_name="s")
    sup = 2000 if full_n else SUP
    cpo = 400 if full_n else CPO
    alloc = 100096 if full_n else HALF_P
    out_shape = (NC, N, hid) if full_n else (N, hid)
    f = pl.kernel(
        _make_seg_body(hid, full_n),
        out_type=jax.ShapeDtypeStruct(out_shape, jnp.float32),
        mesh=mesh,
        scratch_types=(
            [pltpu.VMEM((sup,), jnp.int32)] * 2
            + [pltpu.VMEM((CH, hid), jnp.float32)] * NBUF
            + [pltpu.VMEM((CH,), jnp.int32)] * NBUF
            + [pltpu.VMEM((cpo, hid), jnp.float32)]
            + [pltpu.VMEM_SHARED((alloc, hid), jnp.float32)]
            + [pltpu.SemaphoreType.DMA] * (2 * NBUF)
        ),
        compiler_params=pltpu.CompilerParams(use_tc_tiling_on_sc=False),
    )
    return f(h, src, dst)


BN = 2000


def _layer1_body(x_ref, sa_ref, sb_ref, w0_ref, w1_ref, b1_ref, o_ref):
    x1 = jnp.maximum(
        jnp.dot(x_ref[...] + sa_ref[...] + sb_ref[...], w0_ref[...],
                preferred_element_type=jnp.float32), 0.0)
    o_ref[...] = jnp.dot(x1, w1_ref[...],
                         preferred_element_type=jnp.float32) + b1_ref[...]


def _layer1_tc(x_ext, s0a, s0b, w0ext, w1, b1):
    return pl.pallas_call(
        _layer1_body,
        grid=(N // BN,),
        in_specs=[
            pl.BlockSpec((BN, FIN), lambda i: (i, 0)),
            pl.BlockSpec((BN, FIN), lambda i: (i, 0)),
            pl.BlockSpec((BN, FIN), lambda i: (i, 0)),
            pl.BlockSpec((FIN, HID), lambda i: (0, 0)),
            pl.BlockSpec((HID, HID), lambda i: (0, 0)),
            pl.BlockSpec((1, HID), lambda i: (0, 0)),
        ],
        out_specs=pl.BlockSpec((BN, HID), lambda i: (i, 0)),
        out_shape=jax.ShapeDtypeStruct((N, HID), jnp.float32),
    )(x_ext, s0a, s0b, w0ext, w1, b1)


def _gconv_out_body(h_ref, a_ref, w_ref, b_ref, o_ref):
    x = jnp.maximum(h_ref[...] + a_ref[...], 0.0)
    o_ref[...] = jnp.dot(x, w_ref[...],
                         preferred_element_type=jnp.float32) + b_ref[...]


def _relu_add_matmul(h, a, w, b):
    k = w.shape[0]
    m = w.shape[1]
    return pl.pallas_call(
        _gconv_out_body,
        grid=(N // BN,),
        in_specs=[
            pl.BlockSpec((BN, k), lambda i: (i, 0)),
            pl.BlockSpec((BN, k), lambda i: (i, 0)),
            pl.BlockSpec((k, m), lambda i: (0, 0)),
            pl.BlockSpec((1, m), lambda i: (0, 0)),
        ],
        out_specs=pl.BlockSpec((BN, m), lambda i: (i, 0)),
        out_shape=jax.ShapeDtypeStruct((N, m), jnp.float32),
    )(h, a, w, b)


def kernel(position, velocity, force, edge_index, W0, b0, W1, b1, Wl, bl):
    ones = jnp.ones((N, 1), jnp.float32)
    zpad = jnp.zeros((N, FIN - 10), jnp.float32)
    x_ext = jnp.concatenate([position, velocity, force, ones, zpad], axis=1)
    w0ext = jnp.concatenate(
        [W0, b0[None, :], jnp.zeros((FIN - 10, HID), jnp.float32)], axis=0)
    src = edge_index[0]
    dst = edge_index[1]
    s0 = _segment_sum_sc(x_ext, src, dst, FIN, True)
    h1 = _layer1_tc(x_ext, s0[0], s0[1], w0ext, W1, b1.reshape(1, HID))
    a1 = _segment_sum_sc(h1, src, dst, HID, False)
    return _relu_add_matmul(h1, a1, Wl, bl.reshape(1, bl.shape[0]))
